# baseline (speedup 1.0000x reference)
import functools

import jax
import jax.numpy as jnp
from jax import lax
from jax.experimental import pallas as pl
from jax.experimental.pallas import tpu as pltpu
from jax.experimental.pallas import tpu_sc as plsc

B = 100000
D = 128
DW = D // 2
S = 10
E = 128

NC, NS = 2, 16
NW = NC * NS
BP = 100352
BPW = BP // NW
C = 32
NCHUNK = BPW // C
NR = S * C
_STREAMS = [(0, 128), (128, 128), (256, 64)]

TC_TILE = 2048


def _sc_body(feat_hbm, idx_hbm, outs_hbm, outn_hbm, idx0, idx1, sbuf0, sbuf1,
             nbuf0, nbuf1, mbuf0, mbuf1,
             gsem0, gsem1, ssem0, ssem1, msem0, msem1):
    wid = lax.axis_index("s") * NC + lax.axis_index("c")
    base = wid * BPW
    idxs = (idx0, idx1)
    sbufs = (sbuf0, sbuf1)
    nbufs = (nbuf0, nbuf1)
    mbufs = (mbuf0, mbuf1)
    gsems = (gsem0, gsem1)
    ssems = (ssem0, ssem1)
    msems = (msem0, msem1)

    def self_store_wait(b):
        pltpu.make_async_copy(
            sbufs[b], outs_hbm.at[pl.ds(base, C)], ssems[b]).wait()

    def sum_store_wait(b):
        pltpu.make_async_copy(
            mbufs[b], outn_hbm.at[pl.ds(base, C)], msems[b]).wait()

    def fire(c, b, w):
        ib = (base // C + c) * (S + 1) * C
        pltpu.sync_copy(idx_hbm.at[pl.ds(ib, (S + 1) * C)], idxs[b])

        @pl.when(w)
        def _():
            self_store_wait(b)
        pltpu.async_copy(feat_hbm.at[idxs[b].at[pl.ds(0, C)]], sbufs[b], gsems[b])
        for o, n in _STREAMS:
            pltpu.async_copy(
                feat_hbm.at[idxs[b].at[pl.ds(C + o, n)]],
                nbufs[b].at[pl.ds(o, n)], gsems[b])

    def drain(b):
        pltpu.make_async_copy(
            feat_hbm.at[idxs[b].at[pl.ds(0, C)]], sbufs[b], gsems[b]).wait()
        for o, n in _STREAMS:
            pltpu.make_async_copy(
                feat_hbm.at[idxs[b].at[pl.ds(C + o, n)]],
                nbufs[b].at[pl.ds(o, n)], gsems[b]).wait()

    def self_store(c, b):
        pltpu.async_copy(
            sbufs[b], outs_hbm.at[pl.ds(base + c * C, C)], ssems[b])

    def compute_store(c, b, w):
        nbuf = nbufs[b]
        mbuf = mbufs[b]

        @pl.when(w)
        def _():
            sum_store_wait(b)

        hi_mask = jnp.int32(-65536)

        def node(i, _):
            for q in range(DW // 16):
                sl = pl.ds(16 * q, 16)
                w = nbuf[i, sl]
                acc_lo = lax.bitcast_convert_type(w << 16, jnp.float32)
                acc_hi = lax.bitcast_convert_type(w & hi_mask, jnp.float32)
                for s in range(1, S):
                    w = nbuf[s * C + i, sl]
                    acc_lo = acc_lo + lax.bitcast_convert_type(w << 16, jnp.float32)
                    acc_hi = acc_hi + lax.bitcast_convert_type(w & hi_mask, jnp.float32)
                lo_b = lax.shift_right_logical(
                    lax.bitcast_convert_type(acc_lo, jnp.int32), 16)
                hi_b = lax.bitcast_convert_type(acc_hi, jnp.int32) & hi_mask
                mbuf[i, sl] = hi_b | lo_b
            return 0

        lax.fori_loop(0, C, node, 0)
        pltpu.async_copy(
            mbuf, outn_hbm.at[pl.ds(base + c * C, C)], msems[b])

    fire(0, 0, False)

    def pair(t, _):
        fire(2 * t + 1, 1, t > 0)
        drain(0)
        self_store(2 * t, 0)
        compute_store(2 * t, 0, t > 0)

        @pl.when(t < NCHUNK // 2 - 1)
        def _():
            fire(2 * t + 2, 0, True)
        drain(1)
        self_store(2 * t + 1, 1)
        compute_store(2 * t + 1, 1, t > 0)
        return 0

    lax.fori_loop(0, NCHUNK // 2, pair, 0)
    for b in (0, 1):
        self_store_wait(b)
        sum_store_wait(b)


@functools.partial(
    pl.kernel,
    out_type=(
        jax.ShapeDtypeStruct((BP, DW), jnp.int32),
        jax.ShapeDtypeStruct((BP, DW), jnp.int32),
    ),
    mesh=plsc.VectorSubcoreMesh(
        core_axis_name="c", subcore_axis_name="s", num_cores=NC, num_subcores=NS
    ),
    compiler_params=pltpu.CompilerParams(use_tc_tiling_on_sc=False),
    scratch_types=[
        pltpu.VMEM(((S + 1) * C,), jnp.int32),
        pltpu.VMEM(((S + 1) * C,), jnp.int32),
        pltpu.VMEM((C, DW), jnp.int32),
        pltpu.VMEM((C, DW), jnp.int32),
        pltpu.VMEM((NR, DW), jnp.int32),
        pltpu.VMEM((NR, DW), jnp.int32),
        pltpu.VMEM((C, DW), jnp.int32),
        pltpu.VMEM((C, DW), jnp.int32),
        pltpu.SemaphoreType.DMA,
        pltpu.SemaphoreType.DMA,
        pltpu.SemaphoreType.DMA,
        pltpu.SemaphoreType.DMA,
        pltpu.SemaphoreType.DMA,
        pltpu.SemaphoreType.DMA,
    ],
)
def _sc_gather(feat_hbm, idx_hbm, outs_hbm, outn_hbm, idx0, idx1, sbuf0, sbuf1,
               nbuf0, nbuf1, mbuf0, mbuf1,
               gsem0, gsem1, ssem0, ssem1, msem0, msem1):
    _sc_body(feat_hbm, idx_hbm, outs_hbm, outn_hbm, idx0, idx1, sbuf0, sbuf1,
             nbuf0, nbuf1, mbuf0, mbuf1,
             gsem0, gsem1, ssem0, ssem1, msem0, msem1)


def _mm_body(self_ref, sum_ref, w1_ref, w2_ref, out_ref):
    acc = lax.dot_general(
        w1_ref[...], self_ref[...],
        dimension_numbers=(((1,), (1,)), ((), ())),
        preferred_element_type=jnp.float32,
    )
    acc += lax.dot_general(
        w2_ref[...], sum_ref[...],
        dimension_numbers=(((1,), (1,)), ((), ())),
        preferred_element_type=jnp.float32,
    )
    out_ref[...] = jnp.maximum(acc, 0.0)


def _tc_matmul(self_bf, sum_bf, w1, w2):
    return pl.pallas_call(
        _mm_body,
        grid=(BP // TC_TILE,),
        in_specs=[
            pl.BlockSpec((TC_TILE, D), lambda i: (i, 0)),
            pl.BlockSpec((TC_TILE, D), lambda i: (i, 0)),
            pl.BlockSpec((E, D), lambda i: (0, 0)),
            pl.BlockSpec((E, D), lambda i: (0, 0)),
        ],
        out_specs=pl.BlockSpec((E, TC_TILE), lambda i: (0, i)),
        out_shape=jax.ShapeDtypeStruct((E, BP), jnp.float32),
    )(self_bf, sum_bf, w1, w2)


def _unpack_bf16(x_i32):
    return lax.bitcast_convert_type(x_i32, jnp.bfloat16).reshape(x_i32.shape[0], D)


def kernel(nodes, neigh_idx, features, W):
    nodes = nodes.astype(jnp.int32)
    neigh = neigh_idx.astype(jnp.int32)
    idx_all = jnp.concatenate([nodes[None, :], neigh.T], axis=0)
    idx_all = jnp.pad(idx_all, ((0, 0), (0, BP - B)))
    idx_flat = (idx_all.reshape(S + 1, NW * NCHUNK, C)
                .transpose(1, 0, 2).reshape(-1))
    feat_p = lax.bitcast_convert_type(
        features.astype(jnp.bfloat16).reshape(-1, DW, 2), jnp.int32)
    self_p, sum_p = _sc_gather(feat_p, idx_flat)
    w1 = W[:, :D].astype(jnp.bfloat16)
    w2 = (W[:, D:] * (1.0 / S)).astype(jnp.bfloat16)
    out = _tc_matmul(_unpack_bf16(self_p), _unpack_bf16(sum_p), w1, w2)
    return out[:, :B]


# speedup vs baseline: 1.0943x; 1.0943x over previous
import functools

import jax
import jax.numpy as jnp
from jax import lax
from jax.experimental import pallas as pl
from jax.experimental.pallas import tpu as pltpu
from jax.experimental.pallas import tpu_sc as plsc

B = 100000
D = 128
DW = D // 2
S = 10
E = 128

NC, NS = 2, 16
NW = NC * NS
BP = 100352
BPW = BP // NW
C = 32
NCHUNK = BPW // C
NR = S * C
_STREAMS = [(0, 128), (128, 128), (256, 64)]

TC_TILE = 2048


def _sc_body(feat_hbm, idx_hbm, out_hbm, idx0, idx1, sbuf0, sbuf1,
             nbuf0, nbuf1, mbuf0, mbuf1,
             gsem0, gsem1, ssem0, ssem1, msem0, msem1):
    wid = lax.axis_index("s") * NC + lax.axis_index("c")
    base = wid * BPW
    idxs = (idx0, idx1)
    sbufs = (sbuf0, sbuf1)
    nbufs = (nbuf0, nbuf1)
    mbufs = (mbuf0, mbuf1)
    gsems = (gsem0, gsem1)
    ssems = (ssem0, ssem1)
    msems = (msem0, msem1)

    def self_store_wait(b):
        pltpu.make_async_copy(
            sbufs[b], out_hbm.at[pl.ds(base, C), pl.ds(0, DW)], ssems[b]).wait()

    def sum_store_wait(b):
        pltpu.make_async_copy(
            mbufs[b], out_hbm.at[pl.ds(base, C), pl.ds(DW, DW)], msems[b]).wait()

    def fire(c, b, w):
        ib = (base // C + c) * (S + 1) * C
        pltpu.sync_copy(idx_hbm.at[pl.ds(ib, (S + 1) * C)], idxs[b])

        @pl.when(w)
        def _():
            self_store_wait(b)
        pltpu.async_copy(feat_hbm.at[idxs[b].at[pl.ds(0, C)]], sbufs[b], gsems[b])
        for o, n in _STREAMS:
            pltpu.async_copy(
                feat_hbm.at[idxs[b].at[pl.ds(C + o, n)]],
                nbufs[b].at[pl.ds(o, n)], gsems[b])

    def drain(b):
        pltpu.make_async_copy(
            feat_hbm.at[idxs[b].at[pl.ds(0, C)]], sbufs[b], gsems[b]).wait()
        for o, n in _STREAMS:
            pltpu.make_async_copy(
                feat_hbm.at[idxs[b].at[pl.ds(C + o, n)]],
                nbufs[b].at[pl.ds(o, n)], gsems[b]).wait()

    def self_store(c, b):
        pltpu.async_copy(
            sbufs[b], out_hbm.at[pl.ds(base + c * C, C), pl.ds(0, DW)], ssems[b])

    def compute_store(c, b, w):
        nbuf = nbufs[b]
        mbuf = mbufs[b]

        @pl.when(w)
        def _():
            sum_store_wait(b)

        hi_mask = jnp.int32(-65536)

        def node(i, _):
            for q in range(DW // 16):
                sl = pl.ds(16 * q, 16)
                w = nbuf[i, sl]
                acc_lo = lax.bitcast_convert_type(w << 16, jnp.float32)
                acc_hi = lax.bitcast_convert_type(w & hi_mask, jnp.float32)
                for s in range(1, S):
                    w = nbuf[s * C + i, sl]
                    acc_lo = acc_lo + lax.bitcast_convert_type(w << 16, jnp.float32)
                    acc_hi = acc_hi + lax.bitcast_convert_type(w & hi_mask, jnp.float32)
                lo_b = lax.shift_right_logical(
                    lax.bitcast_convert_type(acc_lo, jnp.int32), 16)
                hi_b = lax.bitcast_convert_type(acc_hi, jnp.int32) & hi_mask
                mbuf[i, sl] = hi_b | lo_b
            return 0

        lax.fori_loop(0, C, node, 0)
        pltpu.async_copy(
            mbuf, out_hbm.at[pl.ds(base + c * C, C), pl.ds(DW, DW)], msems[b])

    fire(0, 0, False)

    def pair(t, _):
        fire(2 * t + 1, 1, t > 0)
        drain(0)
        self_store(2 * t, 0)
        compute_store(2 * t, 0, t > 0)

        @pl.when(t < NCHUNK // 2 - 1)
        def _():
            fire(2 * t + 2, 0, True)
        drain(1)
        self_store(2 * t + 1, 1)
        compute_store(2 * t + 1, 1, t > 0)
        return 0

    lax.fori_loop(0, NCHUNK // 2, pair, 0)
    for b in (0, 1):
        self_store_wait(b)
        sum_store_wait(b)


@functools.partial(
    pl.kernel,
    out_type=jax.ShapeDtypeStruct((BP, 2 * DW), jnp.int32),
    mesh=plsc.VectorSubcoreMesh(
        core_axis_name="c", subcore_axis_name="s", num_cores=NC, num_subcores=NS
    ),
    compiler_params=pltpu.CompilerParams(use_tc_tiling_on_sc=False),
    scratch_types=[
        pltpu.VMEM(((S + 1) * C,), jnp.int32),
        pltpu.VMEM(((S + 1) * C,), jnp.int32),
        pltpu.VMEM((C, DW), jnp.int32),
        pltpu.VMEM((C, DW), jnp.int32),
        pltpu.VMEM((NR, DW), jnp.int32),
        pltpu.VMEM((NR, DW), jnp.int32),
        pltpu.VMEM((C, DW), jnp.int32),
        pltpu.VMEM((C, DW), jnp.int32),
        pltpu.SemaphoreType.DMA,
        pltpu.SemaphoreType.DMA,
        pltpu.SemaphoreType.DMA,
        pltpu.SemaphoreType.DMA,
        pltpu.SemaphoreType.DMA,
        pltpu.SemaphoreType.DMA,
    ],
)
def _sc_gather(feat_hbm, idx_hbm, out_hbm, idx0, idx1, sbuf0, sbuf1,
               nbuf0, nbuf1, mbuf0, mbuf1,
               gsem0, gsem1, ssem0, ssem1, msem0, msem1):
    _sc_body(feat_hbm, idx_hbm, out_hbm, idx0, idx1, sbuf0, sbuf1,
             nbuf0, nbuf1, mbuf0, mbuf1,
             gsem0, gsem1, ssem0, ssem1, msem0, msem1)


def _mm_body(comb_ref, w_ref, out_ref):
    acc = lax.dot_general(
        w_ref[...], comb_ref[...],
        dimension_numbers=(((1,), (1,)), ((), ())),
        preferred_element_type=jnp.float32,
    )
    out_ref[...] = jnp.maximum(acc, 0.0)


def _tc_matmul(comb_bf, w):
    return pl.pallas_call(
        _mm_body,
        grid=(BP // TC_TILE,),
        in_specs=[
            pl.BlockSpec((TC_TILE, 2 * D), lambda i: (i, 0)),
            pl.BlockSpec((E, 2 * D), lambda i: (0, 0)),
        ],
        out_specs=pl.BlockSpec((E, TC_TILE), lambda i: (0, i)),
        out_shape=jax.ShapeDtypeStruct((E, B), jnp.float32),
    )(comb_bf, w)


def kernel(nodes, neigh_idx, features, W):
    nodes = nodes.astype(jnp.int32)
    neigh = neigh_idx.astype(jnp.int32)
    idx_all = jnp.concatenate([nodes[None, :], neigh.T], axis=0)
    idx_all = jnp.pad(idx_all, ((0, 0), (0, BP - B)))
    idx_flat = (idx_all.reshape(S + 1, NW * NCHUNK, C)
                .transpose(1, 0, 2).reshape(-1))
    feat_p = lax.bitcast_convert_type(
        features.astype(jnp.bfloat16).reshape(-1, DW, 2), jnp.int32)
    comb_p = _sc_gather(feat_p, idx_flat)
    comb_bf = lax.bitcast_convert_type(comb_p, jnp.bfloat16).reshape(BP, 2 * D)
    w_mod = jnp.concatenate(
        [W[:, :D], W[:, D:] * (1.0 / S)], axis=1).astype(jnp.bfloat16)
    return _tc_matmul(comb_bf, w_mod)


# speedup vs baseline: 2.4662x; 2.2537x over previous
import functools

import jax
import jax.numpy as jnp
from jax import lax
from jax.experimental import pallas as pl
from jax.experimental.pallas import tpu as pltpu
from jax.experimental.pallas import tpu_sc as plsc

B = 100000
D = 128
S = 10
E = 128

NC, NS = 2, 16
NW = NC * NS
BP = 100352
BPW = BP // NW
C = 32
NCHUNK = BPW // C
NR = S * C
_STREAMS = [(0, 128), (128, 128), (256, 64)]

TC_TILE = 2048


def _sc_body(feat_hbm, idx_hbm, out_hbm, idx0, idx1, sbuf0, sbuf1,
             nbuf0, nbuf1, mbuf0, mbuf1,
             gsem0, gsem1, ssem0, ssem1, msem0, msem1, isem0, isem1):
    wid = lax.axis_index("s") * NC + lax.axis_index("c")
    base = wid * BPW
    idxs = (idx0, idx1)
    sbufs = (sbuf0, sbuf1)
    nbufs = (nbuf0, nbuf1)
    mbufs = (mbuf0, mbuf1)
    gsems = (gsem0, gsem1)
    ssems = (ssem0, ssem1)
    msems = (msem0, msem1)
    isems = (isem0, isem1)

    def self_store_wait(b):
        pltpu.make_async_copy(
            sbufs[b], out_hbm.at[pl.ds(base, C), pl.ds(0, D)], ssems[b]).wait()

    def mean_store_wait(b):
        pltpu.make_async_copy(
            mbufs[b], out_hbm.at[pl.ds(base, C), pl.ds(D, D)], msems[b]).wait()

    def idx_fetch(c, b):
        ib = (base // C + c) * (S + 1) * C
        pltpu.async_copy(idx_hbm.at[pl.ds(ib, (S + 1) * C)], idxs[b], isems[b])

    def idx_wait(b):
        pltpu.make_async_copy(
            idx_hbm.at[pl.ds(0, (S + 1) * C)], idxs[b], isems[b]).wait()

    def fire(c, b, w):
        @pl.when(w)
        def _():
            self_store_wait(b)
        pltpu.async_copy(feat_hbm.at[idxs[b].at[pl.ds(0, C)]], sbufs[b], gsems[b])
        for o, n in _STREAMS:
            pltpu.async_copy(
                feat_hbm.at[idxs[b].at[pl.ds(C + o, n)]],
                nbufs[b].at[pl.ds(o, n)], gsems[b])

    def drain(b):
        pltpu.make_async_copy(
            feat_hbm.at[pl.ds(0, C)], sbufs[b], gsems[b]).wait()
        pltpu.make_async_copy(
            feat_hbm.at[pl.ds(0, NR)], nbufs[b], gsems[b]).wait()

    def self_store(c, b):
        pltpu.async_copy(
            sbufs[b], out_hbm.at[pl.ds(base + c * C, C), pl.ds(0, D)], ssems[b])

    def compute_store(c, b, w):
        nbuf = nbufs[b]
        mbuf = mbufs[b]

        @pl.when(w)
        def _():
            mean_store_wait(b)

        def node(i, _):
            for j in range(D // 16):
                sl = pl.ds(16 * j, 16)
                acc = nbuf[i, sl]
                for s in range(1, S):
                    acc = acc + nbuf[s * C + i, sl]
                mbuf[i, sl] = acc * (1.0 / S)
            return 0

        lax.fori_loop(0, C, node, 0)
        pltpu.async_copy(
            mbuf, out_hbm.at[pl.ds(base + c * C, C), pl.ds(D, D)], msems[b])

    idx_fetch(0, 0)
    idx_wait(0)
    fire(0, 0, False)
    idx_fetch(1, 1)

    def pair(t, _):
        idx_wait(1)
        fire(2 * t + 1, 1, t > 0)
        drain(0)
        self_store(2 * t, 0)

        @pl.when(t < NCHUNK // 2 - 1)
        def _():
            idx_fetch(2 * t + 2, 0)
        compute_store(2 * t, 0, t > 0)

        @pl.when(t < NCHUNK // 2 - 1)
        def _():
            idx_wait(0)
            fire(2 * t + 2, 0, True)
        drain(1)
        self_store(2 * t + 1, 1)

        @pl.when(t < NCHUNK // 2 - 1)
        def _():
            idx_fetch(2 * t + 3, 1)
        compute_store(2 * t + 1, 1, t > 0)
        return 0

    lax.fori_loop(0, NCHUNK // 2, pair, 0)
    for b in (0, 1):
        self_store_wait(b)
        mean_store_wait(b)


@functools.partial(
    pl.kernel,
    out_type=jax.ShapeDtypeStruct((BP, 2 * D), jnp.float32),
    mesh=plsc.VectorSubcoreMesh(
        core_axis_name="c", subcore_axis_name="s", num_cores=NC, num_subcores=NS
    ),
    scratch_types=[
        pltpu.VMEM(((S + 1) * C,), jnp.int32),
        pltpu.VMEM(((S + 1) * C,), jnp.int32),
        pltpu.VMEM((C, D), jnp.float32),
        pltpu.VMEM((C, D), jnp.float32),
        pltpu.VMEM((NR, D), jnp.float32),
        pltpu.VMEM((NR, D), jnp.float32),
        pltpu.VMEM((C, D), jnp.float32),
        pltpu.VMEM((C, D), jnp.float32),
        pltpu.SemaphoreType.DMA,
        pltpu.SemaphoreType.DMA,
        pltpu.SemaphoreType.DMA,
        pltpu.SemaphoreType.DMA,
        pltpu.SemaphoreType.DMA,
        pltpu.SemaphoreType.DMA,
        pltpu.SemaphoreType.DMA,
        pltpu.SemaphoreType.DMA,
    ],
)
def _sc_gather(feat_hbm, idx_hbm, out_hbm, idx0, idx1, sbuf0, sbuf1,
               nbuf0, nbuf1, mbuf0, mbuf1,
               gsem0, gsem1, ssem0, ssem1, msem0, msem1, isem0, isem1):
    _sc_body(feat_hbm, idx_hbm, out_hbm, idx0, idx1, sbuf0, sbuf1,
             nbuf0, nbuf1, mbuf0, mbuf1,
             gsem0, gsem1, ssem0, ssem1, msem0, msem1, isem0, isem1)


def _mm_body(comb_ref, w_ref, out_ref):
    acc = lax.dot_general(
        w_ref[...], comb_ref[...],
        dimension_numbers=(((1,), (1,)), ((), ())),
        preferred_element_type=jnp.float32,
    )
    out_ref[...] = jnp.maximum(acc, 0.0)


def _tc_matmul(combined, w):
    return pl.pallas_call(
        _mm_body,
        grid=(BP // TC_TILE,),
        in_specs=[
            pl.BlockSpec((TC_TILE, 2 * D), lambda i: (i, 0)),
            pl.BlockSpec((E, 2 * D), lambda i: (0, 0)),
        ],
        out_specs=pl.BlockSpec((E, TC_TILE), lambda i: (0, i)),
        out_shape=jax.ShapeDtypeStruct((E, B), jnp.float32),
    )(combined, w)


def kernel(nodes, neigh_idx, features, W):
    nodes = nodes.astype(jnp.int32)
    neigh = neigh_idx.astype(jnp.int32)
    idx_all = jnp.concatenate([nodes[None, :], neigh.T], axis=0)
    idx_all = jnp.pad(idx_all, ((0, 0), (0, BP - B)))
    idx_flat = (idx_all.reshape(S + 1, NW * NCHUNK, C)
                .transpose(1, 0, 2).reshape(-1))
    combined = _sc_gather(features, idx_flat)
    return _tc_matmul(combined, W)
